# Initial kernel scaffold; baseline (speedup 1.0000x reference)
import jax
import jax.numpy as jnp
from jax.experimental import pallas as pl


def kernel(feature, idx1, idx2):
    raise NotImplementedError("write your pallas kernel here")


# speedup vs baseline: 3.2306x; 3.2306x over previous
import jax
import jax.numpy as jnp
import numpy as np
from jax import lax
from jax.experimental import pallas as pl
from jax.experimental.pallas import tpu as pltpu, tpu_sc as plsc

import functools


_LANES = 16


def _make_sc_gather(n_layers, B, S, D):
    n_rows = 3 * B
    assert n_rows <= _LANES
    base = (n_layers - 1) * B * S

    lanes = np.arange(_LANES)
    bat = np.minimum(lanes // 3, B - 1)
    base_np = (base + bat * S).astype(np.int32)

    mesh = plsc.VectorSubcoreMesh(core_axis_name="c", subcore_axis_name="s")

    @functools.partial(
        pl.kernel,
        mesh=mesh,
        out_type=jax.ShapeDtypeStruct((_LANES, D), jnp.float32),
        scratch_types=[
            pltpu.VMEM((_LANES,), jnp.int32),
            pltpu.VMEM((_LANES,), jnp.int32),
            pltpu.VMEM((_LANES,), jnp.int32),
            pltpu.VMEM((_LANES, D), jnp.float32),
            pltpu.SemaphoreType.DMA,
        ],
    )
    def sc_gather(table_hbm, pos_hbm, base_hbm, out_hbm,
                  pos_v, base_v, row_idx, rows, sem):
        wid = lax.axis_index("s") * 2 + lax.axis_index("c")

        @pl.when(wid == 0)
        def _():
            pltpu.sync_copy(pos_hbm, pos_v)
            pltpu.sync_copy(base_hbm, base_v)
            row_idx[...] = base_v[...] + pos_v[...]
            pltpu.async_copy(table_hbm.at[row_idx], rows, sem).wait()
            pltpu.sync_copy(rows, out_hbm)

    return sc_gather, base_np


def kernel(feature, idx1, idx2):
    n_layers, B, S, D = feature.shape
    table = feature.reshape(n_layers * B * S, D)
    zero = jnp.zeros_like(idx1, dtype=jnp.int32)
    pos = jnp.stack([zero, idx1.astype(jnp.int32), idx2.astype(jnp.int32)],
                    axis=1).reshape(-1)
    pos = jnp.concatenate([pos, jnp.zeros((_LANES - 3 * B,), jnp.int32)])
    sc_gather, base_np = _make_sc_gather(n_layers, B, S, D)
    out = sc_gather(table, pos, jnp.asarray(base_np))
    return out[: 3 * B].reshape(B, 3 * D)
